# baseline (speedup 1.0000x reference)
import functools

import jax
import jax.numpy as jnp
from jax import lax
from jax.experimental import pallas as pl
from jax.experimental.pallas import tpu as pltpu
from jax.experimental.pallas import tpu_sc as plsc

N = 10000
E = 160000
D = 256
PATCH_DIM = 32
CONCAT = D + PATCH_DIM
PPS = 4
TEMPERATURE = 0.1
MAX_DIST_RATIO = 0.5

BN = 512
EP = 163840
NPAD = 10008
BE = 2048
CH = 128
NTILES = 32
EPT_G = EP // NTILES
EPT_S = EP // 16
NROW_T = 624
TSW = 384
TDW = 128


def _dense_node_body(pos_ref, pf_ref, pxpy_ref, emb_ref, wpre_ref, bpre_ref,
                     wlin_ref, wsrc_ref, wdst_ref, watt1_ref, watt2_ref,
                     batt2_ref, cpack_ref,
                     ts_ref, td_ref, wm0_ref):
    f32 = jnp.float32
    pos = pos_ref[...]
    x = pos[:, 0:1]
    y = pos[:, 1:2]
    px = pxpy_ref[0:1, :]
    py = pxpy_ref[1:2, :]
    dx = x - px
    dy = y - py
    dist = jnp.sqrt(dx * dx + dy * dy)
    mask = (dist <= MAX_DIST_RATIO).astype(f32)
    w = jnp.exp(-dist / TEMPERATURE) * mask
    ws = jnp.sum(w, axis=1, keepdims=True)
    valid = (ws > 1e-06).astype(f32)
    w = w / (ws + 1e-10)
    contrib = jnp.dot(w, emb_ref[...], preferred_element_type=f32) * valid

    h = jnp.concatenate([pf_ref[...], contrib], axis=1)
    h = jnp.dot(h, wpre_ref[...], preferred_element_type=f32) + bpre_ref[0:1, :]

    x_lin = jnp.dot(h, wlin_ref[...], preferred_element_type=f32)
    a_src = jnp.dot(h, wsrc_ref[...], preferred_element_type=f32)
    a_dst = jnp.dot(h, wdst_ref[...], preferred_element_type=f32)
    b_src = jnp.dot(a_src, watt1_ref[...], preferred_element_type=f32)
    b_dst = jnp.dot(a_dst, watt1_ref[...], preferred_element_type=f32)

    zpad = jnp.zeros((pos.shape[0], 48), f32)
    ts_ref[...] = jnp.concatenate([b_src, pos, zpad, x_lin], axis=1)
    td_ref[...] = jnp.concatenate([b_dst, pos, zpad], axis=1)

    c_self = cpack_ref[1:2, 0:64]
    delta0 = cpack_ref[0:1, :]
    u0 = jax.nn.relu(b_dst - b_src + c_self)
    alpha0 = jnp.dot(u0, watt2_ref[...], preferred_element_type=f32) + batt2_ref[0:1, :]
    w0 = jnp.exp(alpha0)
    m0 = w0 * (x_lin + delta0)
    for c in range(4):
        wm0_ref[c, :, :] = jnp.concatenate(
            [w0[:, 64 * c:64 * c + 64], m0[:, 64 * c:64 * c + 64]], axis=1)


def _dense_node(pos16, pf, pxpy, emb, W_pre, b_pre, W_lin, W_src, W_dst,
                W_att1, W_att2, b_att2, cpack):
    f32 = jnp.float32
    grid = (pl.cdiv(N, BN),)
    full = lambda shape: pl.BlockSpec(shape, lambda i: tuple(0 for _ in shape))
    blk = lambda w: pl.BlockSpec((BN, w), lambda i: (i, 0))
    return pl.pallas_call(
        _dense_node_body,
        grid=grid,
        in_specs=[
            blk(16), blk(D),
            full((8, 16)), full((16, PATCH_DIM)), full((CONCAT, CONCAT)),
            full((8, CONCAT)), full((CONCAT, D)), full((CONCAT, D)),
            full((CONCAT, D)), full((D, 64)), full((64, D)), full((8, D)),
            full((8, D)),
        ],
        out_specs=[blk(TSW), blk(TDW),
                   pl.BlockSpec((4, BN, 128), lambda i: (0, i, 0))],
        out_shape=[
            jax.ShapeDtypeStruct((N, TSW), f32),
            jax.ShapeDtypeStruct((N, TDW), f32),
            jax.ShapeDtypeStruct((4, N, 128), f32),
        ],
    )(pos16, pf, pxpy, emb, W_pre, b_pre, W_lin, W_src, W_dst, W_att1,
      W_att2, b_att2, cpack)


CHG = 80
SBG = 8
ROWS_G = EP // CHG
ROWS_G_T = ROWS_G // NTILES


def _sc_gather_body(srcp2, dstp2, ts_hbm, td_hbm,
                    as_out, ad_out,
                    idxs8, idxd8, bufs0, bufs1, bufd0, bufd1,
                    gs0, gs1, gd0, gd1, os0, os1, od0, od1, six):
    wid = lax.axis_index("c") * 16 + lax.axis_index("s")
    row_t = wid * ROWS_G_T
    bufs = (bufs0, bufs1)
    bufd = (bufd0, bufd1)
    gsem = ((gs0, gd0), (gs1, gd1))
    osem = ((os0, os1), (od0, od1))

    def superblock(sb, _):
        row0 = row_t + sb * SBG
        cA = pltpu.async_copy(srcp2.at[pl.ds(row0, SBG)], idxs8, six)
        cA.wait()
        cB = pltpu.async_copy(dstp2.at[pl.ds(row0, SBG)], idxd8, six)
        cB.wait()
        g = [None] * SBG
        o = [None] * SBG
        for i in range(SBG):
            b = i % 2
            base = (row0 + i) * CHG
            if i >= 2:
                o[i - 2][0].wait()
                o[i - 2][1].wait()
            g[i] = (
                pltpu.async_copy(ts_hbm.at[idxs8.at[i]], bufs[b], gsem[b][0]),
                pltpu.async_copy(td_hbm.at[idxd8.at[i]], bufd[b], gsem[b][1]),
            )
            if i >= 1:
                pb = (i - 1) % 2
                pbase = (row0 + i - 1) * CHG
                g[i - 1][0].wait()
                g[i - 1][1].wait()
                o[i - 1] = (
                    pltpu.async_copy(bufs[pb], as_out.at[pl.ds(pbase, CHG)],
                                     osem[pb][0]),
                    pltpu.async_copy(bufd[pb], ad_out.at[pl.ds(pbase, CHG)],
                                     osem[pb][1]),
                )
        last = SBG - 1
        g[last][0].wait()
        g[last][1].wait()
        lbase = (row0 + last) * CHG
        o[last] = (
            pltpu.async_copy(bufs[last % 2], as_out.at[pl.ds(lbase, CHG)],
                             osem[last % 2][0]),
            pltpu.async_copy(bufd[last % 2], ad_out.at[pl.ds(lbase, CHG)],
                             osem[last % 2][1]),
        )
        o[last - 1][0].wait()
        o[last - 1][1].wait()
        o[last][0].wait()
        o[last][1].wait()
        return 0

    lax.fori_loop(0, ROWS_G_T // SBG, superblock, 0)


def _sc_gather(srcp2, dstp2, ts, td):
    f32 = jnp.float32
    mesh = plsc.VectorSubcoreMesh(core_axis_name="c", subcore_axis_name="s")
    fn = pl.kernel(
        _sc_gather_body,
        out_type=[
            jax.ShapeDtypeStruct((EP, TSW), f32),
            jax.ShapeDtypeStruct((EP, TDW), f32),
        ],
        mesh=mesh,
        scratch_types=[
            pltpu.VMEM((SBG, CHG), jnp.int32),
            pltpu.VMEM((SBG, CHG), jnp.int32),
            pltpu.VMEM((CHG, TSW), f32),
            pltpu.VMEM((CHG, TSW), f32),
            pltpu.VMEM((CHG, TDW), f32),
            pltpu.VMEM((CHG, TDW), f32),
        ] + [pltpu.SemaphoreType.DMA] * 9,
    )
    return fn(srcp2, dstp2, ts, td)


def _edge_mlp_body(as_ref, ad_ref,
                   wp1_ref, bp1_ref, wf_ref, c0_ref, watt2_ref, batt2_ref,
                   wpos2_ref, bpos2_ref,
                   wm_ref):
    f32 = jnp.float32
    bs = as_ref[:, 0:64]
    ps = as_ref[:, 64:80]
    xs = as_ref[:, 128:384]
    bd = ad_ref[:, 0:64]
    pd = ad_ref[:, 64:80]
    rel = pd - ps
    g = bd - bs
    t = jax.nn.relu(jnp.dot(rel, wp1_ref[...], preferred_element_type=f32)
                    + bp1_ref[0:1, :])
    u = jax.nn.relu(g + jnp.dot(t, wf_ref[...], preferred_element_type=f32)
                    + c0_ref[0:1, :])
    alpha = jnp.dot(u, watt2_ref[...], preferred_element_type=f32) + batt2_ref[0:1, :]
    w = jnp.exp(alpha)
    delta = jnp.dot(t, wpos2_ref[...], preferred_element_type=f32) + bpos2_ref[0:1, :]
    m = w * (xs + delta)
    for c in range(4):
        wm_ref[c, :, :] = jnp.concatenate(
            [w[:, 64 * c:64 * c + 64], m[:, 64 * c:64 * c + 64]], axis=1)


def _edge_mlp(a_s, a_d, wp1, bp1, wf, c0, watt2, batt2, wpos2, bpos2):
    f32 = jnp.float32
    grid = (EP // BE,)
    full = lambda shape: pl.BlockSpec(shape, lambda i: tuple(0 for _ in shape))
    return pl.pallas_call(
        _edge_mlp_body,
        grid=grid,
        in_specs=[
            pl.BlockSpec((BE, TSW), lambda i: (i, 0)),
            pl.BlockSpec((BE, TDW), lambda i: (i, 0)),
            full((16, 64)), full((8, 64)), full((64, 64)), full((8, 64)),
            full((64, D)), full((8, D)), full((64, D)), full((8, D)),
        ],
        out_specs=[pl.BlockSpec((4, BE, 128), lambda i: (0, i, 0))],
        out_shape=[jax.ShapeDtypeStruct((4, EP, 128), f32)],
    )(a_s, a_d, wp1, bp1, wf, c0, watt2, batt2, wpos2, bpos2)


def _sc_scatter_body(deff2, wm_hbm, wm0_hbm,
                     acc_out, acc, bufc0, bufc1, idx8,
                     sL0, sL1, sS0, sS1, six):
    core = lax.axis_index("c")
    sub = lax.axis_index("s")
    bufc = (bufc0, bufc1)
    lsem = (sL0, sL1)
    ssem = (sS0, sS1)
    SBS = 8
    rows_t = EPT_S // CH

    for p in range(2):
        cc = core * 2 + p

        @pl.when(sub == 0)
        def _init():
            pltpu.sync_copy(wm0_hbm.at[cc], acc.at[pl.ds(0, N), :])

        plsc.subcore_barrier()

        row_t = sub * rows_t

        def superblock(sb, _):
            row0 = row_t + sb * SBS
            cA = pltpu.async_copy(deff2.at[pl.ds(row0, SBS)], idx8, six)
            cA.wait()
            L = [None] * SBS
            S = [None] * SBS
            for i in range(SBS):
                b = i % 2
                base = (row0 + i) * CH
                if i >= 2:
                    S[i - 2].wait()
                L[i] = pltpu.async_copy(wm_hbm.at[cc, pl.ds(base, CH), :],
                                        bufc[b], lsem[b])
                if i >= 1:
                    pb = (i - 1) % 2
                    L[i - 1].wait()
                    S[i - 1] = pltpu.async_copy(bufc[pb], acc.at[idx8.at[i - 1]],
                                                ssem[pb], add=True)
            L[SBS - 1].wait()
            S[SBS - 1] = pltpu.async_copy(bufc[(SBS - 1) % 2],
                                          acc.at[idx8.at[SBS - 1]],
                                          ssem[(SBS - 1) % 2], add=True)
            S[SBS - 2].wait()
            S[SBS - 1].wait()
            return 0

        lax.fori_loop(0, rows_t // SBS, superblock, 0)

        plsc.subcore_barrier()

        row0 = sub * NROW_T
        pltpu.sync_copy(acc.at[pl.ds(row0, NROW_T), :],
                        acc_out.at[cc, pl.ds(row0, NROW_T), :])

        @pl.when(sub == 15)
        def _tail():
            pltpu.sync_copy(acc.at[pl.ds(16 * NROW_T, N - 16 * NROW_T), :],
                            acc_out.at[cc, pl.ds(16 * NROW_T, N - 16 * NROW_T), :])

        plsc.subcore_barrier()


def _sc_scatter(deff2, wm, wm0):
    f32 = jnp.float32
    mesh = plsc.VectorSubcoreMesh(core_axis_name="c", subcore_axis_name="s")
    fn = pl.kernel(
        _sc_scatter_body,
        out_type=jax.ShapeDtypeStruct((4, N, 128), f32),
        mesh=mesh,
        scratch_types=[
            pltpu.VMEM_SHARED((NPAD, 128), f32),
            pltpu.VMEM((CH, 128), f32),
            pltpu.VMEM((CH, 128), f32),
            pltpu.VMEM((8, CH), jnp.int32),
        ] + [pltpu.SemaphoreType.DMA] * 5,
    )
    return fn(deff2, wm, wm0)


def _div_body(acc_ref, out_ref):
    i = pl.program_id(0)
    den = jnp.concatenate([acc_ref[c, :, 0:64] for c in range(4)], axis=1)
    num = jnp.concatenate([acc_ref[c, :, 64:128] for c in range(4)], axis=1)
    out_ref[...] = num / (den + 1e-16)


def _div(acc):
    f32 = jnp.float32
    grid = (pl.cdiv(N, BN),)
    return pl.pallas_call(
        _div_body, grid=grid,
        in_specs=[pl.BlockSpec((4, BN, 128), lambda i: (0, i, 0))],
        out_specs=pl.BlockSpec((BN, D), lambda i: (i, 0)),
        out_shape=jax.ShapeDtypeStruct((N, D), f32),
    )(acc)


def kernel(point_features, edge_index, point_positions, naip_embeddings,
           naip_bbox, center, scale, W_pre, b_pre, W_lin, W_src, W_dst,
           W_pos1, b_pos1, W_pos2, b_pos2, W_att1, b_att1, W_att2, b_att2):
    f32 = jnp.float32

    minx, miny, maxx, maxy = naip_bbox[0], naip_bbox[1], naip_bbox[2], naip_bbox[3]
    psx = (maxx - minx) / PPS
    psy = (maxy - miny) / PPS
    idx = jnp.arange(PPS, dtype=f32)
    xc = minx + psx / 2.0 + idx * psx
    yc = miny + psy / 2.0 + idx * psy
    gy, gx = jnp.meshgrid(yc, xc, indexing='ij')
    positions = jnp.stack([gx.flatten(), gy.flatten()], axis=1)
    patch_pos = (positions - center[:, :2]) / scale
    pxpy = jnp.zeros((8, 16), f32).at[0].set(patch_pos[:, 0]).at[1].set(patch_pos[:, 1])

    Wf = W_pos2 @ W_att1
    c0 = b_pos2 @ W_att1 + b_att1
    t0 = jax.nn.relu(b_pos1)
    c_self = t0 @ Wf + c0
    delta0 = t0 @ W_pos2 + b_pos2
    cpack = jnp.zeros((8, D), f32).at[0].set(delta0).at[1, 0:64].set(c_self)

    pos16 = jnp.zeros((N, 16), f32).at[:, 0:3].set(point_positions)
    bpre = jnp.zeros((8, CONCAT), f32).at[0].set(b_pre)
    batt2 = jnp.zeros((8, D), f32).at[0].set(b_att2)
    wp1 = jnp.zeros((16, 64), f32).at[0:3, :].set(W_pos1)
    bp1 = jnp.zeros((8, 64), f32).at[0].set(b_pos1)
    c0row = jnp.zeros((8, 64), f32).at[0].set(c0)
    bpos2row = jnp.zeros((8, D), f32).at[0].set(b_pos2)

    src = edge_index[0]
    dst = edge_index[1]
    pad = EP - E
    srcp = jnp.concatenate([src, jnp.zeros((pad,), src.dtype)])
    dstp = jnp.concatenate([dst, jnp.zeros((pad,), dst.dtype)])
    deff = jnp.where(srcp == dstp, N, dstp).astype(jnp.int32)
    deff2 = deff.reshape(EP // CH, CH)
    srcp2 = srcp.astype(jnp.int32).reshape(ROWS_G, CHG)
    dstp2 = dstp.astype(jnp.int32).reshape(ROWS_G, CHG)

    ts, td, wm0 = _dense_node(
        pos16, point_features, pxpy, naip_embeddings, W_pre, bpre, W_lin,
        W_src, W_dst, W_att1, W_att2, batt2, cpack)

    a_s, a_d = _sc_gather(srcp2, dstp2, ts, td)

    wm, = _edge_mlp(a_s, a_d, wp1, bp1, Wf, c0row, W_att2, batt2, W_pos2,
                    bpos2row)

    acc = _sc_scatter(deff2, wm, wm0)

    return _div(acc)


# speedup vs baseline: 1.1181x; 1.1181x over previous
import functools

import jax
import jax.numpy as jnp
from jax import lax
from jax.experimental import pallas as pl
from jax.experimental.pallas import tpu as pltpu
from jax.experimental.pallas import tpu_sc as plsc

N = 10000
E = 160000
D = 256
PATCH_DIM = 32
CONCAT = D + PATCH_DIM
PPS = 4
TEMPERATURE = 0.1
MAX_DIST_RATIO = 0.5

BN = 512
EP = 163840
NPAD = 10008
BE = 2048
CH = 128
NTILES = 32
EPT_G = EP // NTILES
EPT_S = EP // 16
NROW_T = 624
TSW = 128
XLW = 128
TDW = 128


def _dense_node_body(pos_ref, pf_ref, pxpy_ref, emb_ref, wpre_ref, bpre_ref,
                     wlin_ref, wsrc_ref, wdst_ref, watt1_ref, watt2_ref,
                     batt2_ref, cpack_ref,
                     ts_ref, td_ref, xl_ref, wm0_ref):
    f32 = jnp.float32
    pos = pos_ref[...]
    x = pos[:, 0:1]
    y = pos[:, 1:2]
    px = pxpy_ref[0:1, :]
    py = pxpy_ref[1:2, :]
    dx = x - px
    dy = y - py
    dist = jnp.sqrt(dx * dx + dy * dy)
    mask = (dist <= MAX_DIST_RATIO).astype(f32)
    w = jnp.exp(-dist / TEMPERATURE) * mask
    ws = jnp.sum(w, axis=1, keepdims=True)
    valid = (ws > 1e-06).astype(f32)
    w = w / (ws + 1e-10)
    contrib = jnp.dot(w, emb_ref[...], preferred_element_type=f32) * valid

    h = jnp.concatenate([pf_ref[...], contrib], axis=1)
    h = jnp.dot(h, wpre_ref[...], preferred_element_type=f32) + bpre_ref[0:1, :]

    x_lin = jnp.dot(h, wlin_ref[...], preferred_element_type=f32)
    a_src = jnp.dot(h, wsrc_ref[...], preferred_element_type=f32)
    a_dst = jnp.dot(h, wdst_ref[...], preferred_element_type=f32)
    b_src = jnp.dot(a_src, watt1_ref[...], preferred_element_type=f32)
    b_dst = jnp.dot(a_dst, watt1_ref[...], preferred_element_type=f32)

    zpad = jnp.zeros((pos.shape[0], 48), f32)
    ts_ref[...] = jnp.concatenate([b_src, pos, zpad], axis=1)
    td_ref[...] = jnp.concatenate([b_dst, pos, zpad], axis=1)
    xb = x_lin.astype(jnp.bfloat16)
    lo16 = jax.lax.bitcast_convert_type(xb[:, 0:128], jnp.uint16)
    hi16 = jax.lax.bitcast_convert_type(xb[:, 128:256], jnp.uint16)
    xl_ref[...] = lo16.astype(jnp.uint32) | (hi16.astype(jnp.uint32) << 16)

    c_self = cpack_ref[1:2, 0:64]
    delta0 = cpack_ref[0:1, :]
    u0 = jax.nn.relu(b_dst - b_src + c_self)
    alpha0 = jnp.dot(u0, watt2_ref[...], preferred_element_type=f32) + batt2_ref[0:1, :]
    w0 = jnp.exp(alpha0)
    m0 = w0 * (x_lin + delta0)
    for c in range(4):
        wm0_ref[c, :, :] = jnp.concatenate(
            [w0[:, 64 * c:64 * c + 64], m0[:, 64 * c:64 * c + 64]], axis=1)


def _dense_node(pos16, pf, pxpy, emb, W_pre, b_pre, W_lin, W_src, W_dst,
                W_att1, W_att2, b_att2, cpack):
    f32 = jnp.float32
    grid = (pl.cdiv(N, BN),)
    full = lambda shape: pl.BlockSpec(shape, lambda i: tuple(0 for _ in shape))
    blk = lambda w: pl.BlockSpec((BN, w), lambda i: (i, 0))
    return pl.pallas_call(
        _dense_node_body,
        grid=grid,
        in_specs=[
            blk(16), blk(D),
            full((8, 16)), full((16, PATCH_DIM)), full((CONCAT, CONCAT)),
            full((8, CONCAT)), full((CONCAT, D)), full((CONCAT, D)),
            full((CONCAT, D)), full((D, 64)), full((64, D)), full((8, D)),
            full((8, D)),
        ],
        out_specs=[blk(TSW), blk(TDW), blk(XLW),
                   pl.BlockSpec((4, BN, 128), lambda i: (0, i, 0))],
        out_shape=[
            jax.ShapeDtypeStruct((N, TSW), f32),
            jax.ShapeDtypeStruct((N, TDW), f32),
            jax.ShapeDtypeStruct((N, XLW), jnp.uint32),
            jax.ShapeDtypeStruct((4, N, 128), f32),
        ],
    )(pos16, pf, pxpy, emb, W_pre, b_pre, W_lin, W_src, W_dst, W_att1,
      W_att2, b_att2, cpack)


CHG = 80
SBG = 8
ROWS_G = EP // CHG
ROWS_G_T = ROWS_G // NTILES


def _sc_gather_body(srcp2, dstp2, ts_hbm, td_hbm, xl_hbm,
                    as_out, ad_out, ax_out,
                    idxs8, idxd8, bufs0, bufs1, bufd0, bufd1, bufx0, bufx1,
                    gs0, gs1, gd0, gd1, gx0, gx1,
                    os0, os1, od0, od1, ox0, ox1, six):
    wid = lax.axis_index("c") * 16 + lax.axis_index("s")
    row_t = wid * ROWS_G_T
    bufs = (bufs0, bufs1)
    bufd = (bufd0, bufd1)
    bufx = (bufx0, bufx1)
    gsem = ((gs0, gd0, gx0), (gs1, gd1, gx1))
    osem = ((os0, od0, ox0), (os1, od1, ox1))

    def gathers(i, b, row0):
        return (
            pltpu.async_copy(ts_hbm.at[idxs8.at[i]], bufs[b], gsem[b][0]),
            pltpu.async_copy(td_hbm.at[idxd8.at[i]], bufd[b], gsem[b][1]),
            pltpu.async_copy(xl_hbm.at[idxs8.at[i]], bufx[b], gsem[b][2]),
        )

    def stores(i, b, row0):
        base = (row0 + i) * CHG
        return (
            pltpu.async_copy(bufs[b], as_out.at[pl.ds(base, CHG)], osem[b][0]),
            pltpu.async_copy(bufd[b], ad_out.at[pl.ds(base, CHG)], osem[b][1]),
            pltpu.async_copy(bufx[b], ax_out.at[pl.ds(base, CHG)], osem[b][2]),
        )

    def superblock(sb, _):
        row0 = row_t + sb * SBG
        cA = pltpu.async_copy(srcp2.at[pl.ds(row0, SBG)], idxs8, six)
        cA.wait()
        cB = pltpu.async_copy(dstp2.at[pl.ds(row0, SBG)], idxd8, six)
        cB.wait()
        g = [None] * SBG
        o = [None] * SBG
        for i in range(SBG):
            b = i % 2
            if i >= 2:
                for c in o[i - 2]:
                    c.wait()
            g[i] = gathers(i, b, row0)
            if i >= 1:
                for c in g[i - 1]:
                    c.wait()
                o[i - 1] = stores(i - 1, (i - 1) % 2, row0)
        last = SBG - 1
        for c in g[last]:
            c.wait()
        o[last] = stores(last, last % 2, row0)
        for c in o[last - 1]:
            c.wait()
        for c in o[last]:
            c.wait()
        return 0

    lax.fori_loop(0, ROWS_G_T // SBG, superblock, 0)


def _sc_gather(srcp2, dstp2, ts, td, xl):
    f32 = jnp.float32
    bf16 = jnp.bfloat16
    mesh = plsc.VectorSubcoreMesh(core_axis_name="c", subcore_axis_name="s")
    fn = pl.kernel(
        _sc_gather_body,
        out_type=[
            jax.ShapeDtypeStruct((EP, TSW), f32),
            jax.ShapeDtypeStruct((EP, TDW), f32),
            jax.ShapeDtypeStruct((EP, XLW), jnp.uint32),
        ],
        mesh=mesh,
        scratch_types=[
            pltpu.VMEM((SBG, CHG), jnp.int32),
            pltpu.VMEM((SBG, CHG), jnp.int32),
            pltpu.VMEM((CHG, TSW), f32),
            pltpu.VMEM((CHG, TSW), f32),
            pltpu.VMEM((CHG, TDW), f32),
            pltpu.VMEM((CHG, TDW), f32),
            pltpu.VMEM((CHG, XLW), jnp.uint32),
            pltpu.VMEM((CHG, XLW), jnp.uint32),
        ] + [pltpu.SemaphoreType.DMA] * 13,
    )
    return fn(srcp2, dstp2, ts, td, xl)


def _edge_mlp_body(as_ref, ad_ref, ax_ref,
                   wp1_ref, bp1_ref, wf_ref, c0_ref, watt2_ref, batt2_ref,
                   wpos2_ref, bpos2_ref,
                   wm_ref):
    f32 = jnp.float32
    bs = as_ref[:, 0:64]
    ps = as_ref[:, 64:80]
    pk = ax_ref[...]
    lo = jax.lax.bitcast_convert_type((pk & 0xFFFF).astype(jnp.uint16),
                                      jnp.bfloat16).astype(f32)
    hi = jax.lax.bitcast_convert_type((pk >> 16).astype(jnp.uint16),
                                      jnp.bfloat16).astype(f32)
    xs = jnp.concatenate([lo, hi], axis=1)
    bd = ad_ref[:, 0:64]
    pd = ad_ref[:, 64:80]
    rel = pd - ps
    g = bd - bs
    t = jax.nn.relu(jnp.dot(rel, wp1_ref[...], preferred_element_type=f32)
                    + bp1_ref[0:1, :])
    u = jax.nn.relu(g + jnp.dot(t, wf_ref[...], preferred_element_type=f32)
                    + c0_ref[0:1, :])
    alpha = jnp.dot(u, watt2_ref[...], preferred_element_type=f32) + batt2_ref[0:1, :]
    w = jnp.exp(alpha)
    delta = jnp.dot(t, wpos2_ref[...], preferred_element_type=f32) + bpos2_ref[0:1, :]
    m = w * (xs + delta)
    for c in range(4):
        wm_ref[c, :, :] = jnp.concatenate(
            [w[:, 64 * c:64 * c + 64], m[:, 64 * c:64 * c + 64]], axis=1)


def _edge_mlp(a_s, a_d, a_x, wp1, bp1, wf, c0, watt2, batt2, wpos2, bpos2):
    f32 = jnp.float32
    grid = (EP // BE,)
    full = lambda shape: pl.BlockSpec(shape, lambda i: tuple(0 for _ in shape))
    return pl.pallas_call(
        _edge_mlp_body,
        grid=grid,
        in_specs=[
            pl.BlockSpec((BE, TSW), lambda i: (i, 0)),
            pl.BlockSpec((BE, TDW), lambda i: (i, 0)),
            pl.BlockSpec((BE, XLW), lambda i: (i, 0)),
            full((16, 64)), full((8, 64)), full((64, 64)), full((8, 64)),
            full((64, D)), full((8, D)), full((64, D)), full((8, D)),
        ],
        out_specs=[pl.BlockSpec((4, BE, 128), lambda i: (0, i, 0))],
        out_shape=[jax.ShapeDtypeStruct((4, EP, 128), f32)],
    )(a_s, a_d, a_x, wp1, bp1, wf, c0, watt2, batt2, wpos2, bpos2)


def _sc_scatter_body(deff2, wm_hbm, wm0_hbm,
                     acc_out, acc, bufc0, bufc1, idx8,
                     sL0, sL1, sS0, sS1, six):
    core = lax.axis_index("c")
    sub = lax.axis_index("s")
    bufc = (bufc0, bufc1)
    lsem = (sL0, sL1)
    ssem = (sS0, sS1)
    SBS = 8
    rows_t = EPT_S // CH

    for p in range(2):
        cc = core * 2 + p

        @pl.when(sub == 0)
        def _init():
            pltpu.sync_copy(wm0_hbm.at[cc], acc.at[pl.ds(0, N), :])

        plsc.subcore_barrier()

        row_t = sub * rows_t

        def superblock(sb, _):
            row0 = row_t + sb * SBS
            cA = pltpu.async_copy(deff2.at[pl.ds(row0, SBS)], idx8, six)
            cA.wait()
            L = [None] * SBS
            S = [None] * SBS
            for i in range(SBS):
                b = i % 2
                base = (row0 + i) * CH
                if i >= 2:
                    S[i - 2].wait()
                L[i] = pltpu.async_copy(wm_hbm.at[cc, pl.ds(base, CH), :],
                                        bufc[b], lsem[b])
                if i >= 1:
                    pb = (i - 1) % 2
                    L[i - 1].wait()
                    S[i - 1] = pltpu.async_copy(bufc[pb], acc.at[idx8.at[i - 1]],
                                                ssem[pb], add=True)
            L[SBS - 1].wait()
            S[SBS - 1] = pltpu.async_copy(bufc[(SBS - 1) % 2],
                                          acc.at[idx8.at[SBS - 1]],
                                          ssem[(SBS - 1) % 2], add=True)
            S[SBS - 2].wait()
            S[SBS - 1].wait()
            return 0

        lax.fori_loop(0, rows_t // SBS, superblock, 0)

        plsc.subcore_barrier()

        row0 = sub * NROW_T
        pltpu.sync_copy(acc.at[pl.ds(row0, NROW_T), :],
                        acc_out.at[cc, pl.ds(row0, NROW_T), :])

        @pl.when(sub == 15)
        def _tail():
            pltpu.sync_copy(acc.at[pl.ds(16 * NROW_T, N - 16 * NROW_T), :],
                            acc_out.at[cc, pl.ds(16 * NROW_T, N - 16 * NROW_T), :])

        plsc.subcore_barrier()


def _sc_scatter(deff2, wm, wm0):
    f32 = jnp.float32
    mesh = plsc.VectorSubcoreMesh(core_axis_name="c", subcore_axis_name="s")
    fn = pl.kernel(
        _sc_scatter_body,
        out_type=jax.ShapeDtypeStruct((4, N, 128), f32),
        mesh=mesh,
        scratch_types=[
            pltpu.VMEM_SHARED((NPAD, 128), f32),
            pltpu.VMEM((CH, 128), f32),
            pltpu.VMEM((CH, 128), f32),
            pltpu.VMEM((8, CH), jnp.int32),
        ] + [pltpu.SemaphoreType.DMA] * 5,
    )
    return fn(deff2, wm, wm0)


def _div_body(acc_ref, out_ref):
    i = pl.program_id(0)
    den = jnp.concatenate([acc_ref[c, :, 0:64] for c in range(4)], axis=1)
    num = jnp.concatenate([acc_ref[c, :, 64:128] for c in range(4)], axis=1)
    out_ref[...] = num / (den + 1e-16)


def _div(acc):
    f32 = jnp.float32
    grid = (pl.cdiv(N, BN),)
    return pl.pallas_call(
        _div_body, grid=grid,
        in_specs=[pl.BlockSpec((4, BN, 128), lambda i: (0, i, 0))],
        out_specs=pl.BlockSpec((BN, D), lambda i: (i, 0)),
        out_shape=jax.ShapeDtypeStruct((N, D), f32),
    )(acc)


def kernel(point_features, edge_index, point_positions, naip_embeddings,
           naip_bbox, center, scale, W_pre, b_pre, W_lin, W_src, W_dst,
           W_pos1, b_pos1, W_pos2, b_pos2, W_att1, b_att1, W_att2, b_att2):
    f32 = jnp.float32

    minx, miny, maxx, maxy = naip_bbox[0], naip_bbox[1], naip_bbox[2], naip_bbox[3]
    psx = (maxx - minx) / PPS
    psy = (maxy - miny) / PPS
    idx = jnp.arange(PPS, dtype=f32)
    xc = minx + psx / 2.0 + idx * psx
    yc = miny + psy / 2.0 + idx * psy
    gy, gx = jnp.meshgrid(yc, xc, indexing='ij')
    positions = jnp.stack([gx.flatten(), gy.flatten()], axis=1)
    patch_pos = (positions - center[:, :2]) / scale
    pxpy = jnp.zeros((8, 16), f32).at[0].set(patch_pos[:, 0]).at[1].set(patch_pos[:, 1])

    Wf = W_pos2 @ W_att1
    c0 = b_pos2 @ W_att1 + b_att1
    t0 = jax.nn.relu(b_pos1)
    c_self = t0 @ Wf + c0
    delta0 = t0 @ W_pos2 + b_pos2
    cpack = jnp.zeros((8, D), f32).at[0].set(delta0).at[1, 0:64].set(c_self)

    pos16 = jnp.zeros((N, 16), f32).at[:, 0:3].set(point_positions)
    bpre = jnp.zeros((8, CONCAT), f32).at[0].set(b_pre)
    batt2 = jnp.zeros((8, D), f32).at[0].set(b_att2)
    wp1 = jnp.zeros((16, 64), f32).at[0:3, :].set(W_pos1)
    bp1 = jnp.zeros((8, 64), f32).at[0].set(b_pos1)
    c0row = jnp.zeros((8, 64), f32).at[0].set(c0)
    bpos2row = jnp.zeros((8, D), f32).at[0].set(b_pos2)

    src = edge_index[0]
    dst = edge_index[1]
    pad = EP - E
    srcp = jnp.concatenate([src, jnp.zeros((pad,), src.dtype)])
    dstp = jnp.concatenate([dst, jnp.zeros((pad,), dst.dtype)])
    deff = jnp.where(srcp == dstp, N, dstp).astype(jnp.int32)
    deff2 = deff.reshape(EP // CH, CH)
    srcp2 = srcp.astype(jnp.int32).reshape(ROWS_G, CHG)
    dstp2 = dstp.astype(jnp.int32).reshape(ROWS_G, CHG)

    ts, td, xl, wm0 = _dense_node(
        pos16, point_features, pxpy, naip_embeddings, W_pre, bpre, W_lin,
        W_src, W_dst, W_att1, W_att2, batt2, cpack)

    a_s, a_d, a_x = _sc_gather(srcp2, dstp2, ts, td, xl)

    wm, = _edge_mlp(a_s, a_d, a_x, wp1, bp1, Wf, c0row, W_att2, batt2,
                    W_pos2, bpos2row)

    acc = _sc_scatter(deff2, wm, wm0)

    return _div(acc)
